# baseline (speedup 1.0000x reference)
import jax
import jax.numpy as jnp
from jax import lax
from jax.experimental import pallas as pl
from jax.experimental.pallas import tpu as pltpu
from jax.experimental.pallas import tpu_sc as plsc

N = 10000
NP = 10240
NC, NS = 2, 16
NWORK = NC * NS
CHUNK = 128
CH = 80
EPT = CH * CHUNK
ETOT = NWORK * EPT
RPT = NP // NS
DW = 128
PW = 64

NBUF = 8
LAG = 4


def _sc_mesh():
    return plsc.VectorSubcoreMesh(core_axis_name="c", subcore_axis_name="s",
                                  num_cores=NC, num_subcores=NS)


def _deg_body(dst_hbm, ones_hbm, zeros_hbm, out_hbm, dst_v, ones_v, acc,
              *sems):
    c = lax.axis_index("c")
    s = lax.axis_index("s")
    lin = c * NS + s
    pltpu.sync_copy(dst_hbm.at[lin], dst_v)
    pltpu.sync_copy(ones_hbm, ones_v)
    pltpu.sync_copy(zeros_hbm, acc.at[pl.ds(s * RPT, RPT)])
    plsc.subcore_barrier()
    dsc = {}
    for ch in range(CH):
        if ch >= 4:
            dsc[ch - 4].wait()
        dsc[ch] = pltpu.async_copy(ones_v, acc.at[dst_v.at[ch]],
                                   sems[ch % 4], add=True)
    for ch in range(CH - 4, CH):
        dsc[ch].wait()
    plsc.subcore_barrier()
    for j in range(RPT // CHUNK):
        pltpu.sync_copy(acc.at[pl.ds(s * RPT + j * CHUNK, CHUNK)],
                        out_hbm.at[c, s * (RPT // CHUNK) + j])


_sc_cache = {}


def _deg_call(*args):
    if "deg" not in _sc_cache:
        _sc_cache["deg"] = pl.kernel(
            _deg_body,
            out_type=jax.ShapeDtypeStruct((NC, NP // CHUNK, CHUNK),
                                          jnp.float32),
            mesh=_sc_mesh(),
            scratch_types=[
                pltpu.VMEM((CH, CHUNK), jnp.int32),
                pltpu.VMEM((CHUNK,), jnp.float32),
                pltpu.VMEM_SHARED((NP,), jnp.float32),
            ] + [pltpu.SemaphoreType.DMA] * 4,
        )
    return _sc_cache["deg"](*args)


def _prop_body(g_hbm, src_hbm, dst_hbm, zeros_hbm, out_hbm,
               src_v, dst_v, rows, acc, *sems):
    gsem = sems[:NBUF]
    ssem = sems[NBUF:]
    c = lax.axis_index("c")
    s = lax.axis_index("s")
    lin = c * NS + s
    pltpu.sync_copy(src_hbm.at[lin], src_v)
    pltpu.sync_copy(dst_hbm.at[lin], dst_v)
    pltpu.sync_copy(zeros_hbm, acc.at[pl.ds(s * RPT, RPT)])
    plsc.subcore_barrier()
    dg = {}
    dsc = {}
    for j in range(LAG):
        dg[j] = pltpu.async_copy(g_hbm.at[src_v.at[j]],
                                 rows.at[j % NBUF], gsem[j % NBUF])
    for ch in range(CH):
        b = ch % NBUF
        dg[ch].wait()
        dsc[ch] = pltpu.async_copy(rows.at[b], acc.at[dst_v.at[ch]],
                                   ssem[b], add=True)
        g = ch + LAG
        if g < CH:
            bg = g % NBUF
            if g - NBUF >= 0:
                dsc[g - NBUF].wait()
            dg[g] = pltpu.async_copy(g_hbm.at[src_v.at[g]],
                                     rows.at[bg], gsem[bg])
    for ch in range(CH - NBUF, CH):
        dsc[ch].wait()
    plsc.subcore_barrier()
    pltpu.sync_copy(acc.at[pl.ds(s * RPT, RPT)],
                    out_hbm.at[c, pl.ds(s * RPT, RPT)])


def _prop(*args):
    if "prop" not in _sc_cache:
        _sc_cache["prop"] = pl.kernel(
            _prop_body,
            out_type=jax.ShapeDtypeStruct((NC, NP, PW), jnp.float32),
            mesh=_sc_mesh(),
            compiler_params=pltpu.CompilerParams(use_tc_tiling_on_sc=False),
            scratch_types=[
                pltpu.VMEM((CH, CHUNK), jnp.int32),
                pltpu.VMEM((CH, CHUNK), jnp.int32),
                pltpu.VMEM((NBUF, CHUNK, PW), jnp.float32),
                pltpu.VMEM_SHARED((NP, PW), jnp.float32),
            ] + [pltpu.SemaphoreType.DMA] * (2 * NBUF),
        )
    return _sc_cache["prop"](*args)


_BLK = 512
_GRID = (NP // 2) // _BLK
_HOFF = _GRID


def _tc1_body(dl_ref, dh_ref, xl_ref, xh_ref, w_ref, g_ref):
    g_ref[:, :64] = dl_ref[...] * jnp.dot(xl_ref[...], w_ref[...],
                                          preferred_element_type=jnp.float32)
    g_ref[:, 64:] = dh_ref[...] * jnp.dot(xh_ref[...], w_ref[...],
                                          preferred_element_type=jnp.float32)


def _tc_mid_body(dn, acc_ref, g_ref, dl_ref, dh_ref, b_ref, w_ref, out_ref):
    d = w_ref.shape[0]
    for off, dref in ((0, dl_ref), (64, dh_ref)):
        dinv = dref[...]
        h = jnp.maximum(
            dinv * (acc_ref[:, off:off + d] + g_ref[:, off:off + d])
            + b_ref[...], 0.0)
        out_ref[:, off:off + dn] = dinv * jnp.dot(
            h, w_ref[...], preferred_element_type=jnp.float32)
        if dn < 64:
            out_ref[:, off + dn:off + 64] = jnp.zeros((_BLK, 64 - dn),
                                                      jnp.float32)


def _tc_mu_body(acc_ref, g_ref, dl_ref, dh_ref, b_ref, mul_ref, muh_ref):
    mul_ref[...] = (dl_ref[...] * (acc_ref[:, 0:16] + g_ref[:, 0:16])
                    + b_ref[...])
    muh_ref[...] = (dh_ref[...] * (acc_ref[:, 64:80] + g_ref[:, 64:80])
                    + b_ref[...])


_DBLK = 400
_DGRID = N // _DBLK


def _tc_dec_body(mu_i, mu_j, out_ref):
    logits = lax.dot_general(mu_i[...], mu_j[...],
                             (((1,), (1,)), ((), ())),
                             preferred_element_type=jnp.float32)
    out_ref[...] = jax.nn.sigmoid(logits)


def _lo_spec(d):
    return pl.BlockSpec((_BLK, d), lambda i: (i, 0))


def _hi_spec(d):
    return pl.BlockSpec((_BLK, d), lambda i: (i + _HOFF, 0))


def _full_spec(shape):
    return pl.BlockSpec(shape, lambda i: tuple(0 for _ in shape))


def _tc1(dinv, x_p, W1):
    return pl.pallas_call(
        _tc1_body,
        grid=(_GRID,),
        in_specs=[_lo_spec(1), _hi_spec(1), _lo_spec(128), _hi_spec(128),
                  _full_spec((128, 64))],
        out_specs=_lo_spec(128),
        out_shape=jax.ShapeDtypeStruct((NP // 2, 128), jnp.float32),
    )(dinv, dinv, x_p, x_p, W1)


def _tc_mid(accp, gp, dinv, b, W, dn):
    d = W.shape[0]
    return pl.pallas_call(
        lambda *refs: _tc_mid_body(dn, *refs),
        grid=(_GRID,),
        in_specs=[_lo_spec(128), _lo_spec(128), _lo_spec(1), _hi_spec(1),
                  _full_spec((1, d)), _full_spec((d, dn))],
        out_specs=_lo_spec(128),
        out_shape=jax.ShapeDtypeStruct((NP // 2, 128), jnp.float32),
    )(accp, gp, dinv, dinv, b, W)


def _tc_mu(accp, gp, dinv, b):
    return pl.pallas_call(
        _tc_mu_body,
        grid=(_GRID,),
        in_specs=[_lo_spec(128), _lo_spec(128), _lo_spec(1), _hi_spec(1),
                  _full_spec((1, 16))],
        out_specs=[_lo_spec(16), _lo_spec(16)],
        out_shape=[jax.ShapeDtypeStruct((NP // 2, 16), jnp.float32),
                   jax.ShapeDtypeStruct((NP // 2, 16), jnp.float32)],
    )(accp, gp, dinv, dinv, b)


def _tc_dec(mu):
    return pl.pallas_call(
        _tc_dec_body,
        grid=(_DGRID,),
        in_specs=[pl.BlockSpec((_DBLK, 16), lambda i: (i, 0)),
                  pl.BlockSpec((N, 16), lambda i: (0, 0))],
        out_specs=pl.BlockSpec((_DBLK, N), lambda i: (i, 0)),
        out_shape=jax.ShapeDtypeStruct((N, N), jnp.float32),
    )(mu, mu)


def kernel(x, edge_index, W1, b1, W2, b2, Wmu, bmu, Wlv, blv):
    del Wlv, blv
    src = edge_index[0]
    dst = edge_index[1]
    pad = ETOT - src.shape[0]
    fill = jnp.full((pad,), N, dtype=jnp.int32)
    src_n = jnp.concatenate([src, fill])
    dst_n = jnp.concatenate([dst, fill])
    half = NP // 2

    def pos(n):
        return jnp.where(n < half, 2 * n, 2 * (n - half) + 1)

    dst_deg = dst_n.reshape(NWORK, CH, CHUNK)
    srcP = pos(src_n).reshape(NWORK, CH, CHUNK)
    dstP = pos(dst_n).reshape(NWORK, CH, CHUNK)
    x_p = jnp.concatenate(
        [x, jnp.zeros((NP - N, x.shape[1]), jnp.float32)], axis=0)

    ones128 = jnp.ones((CHUNK,), jnp.float32)
    zrow = jnp.zeros((RPT,), jnp.float32)
    zwide = jnp.zeros((RPT, PW), jnp.float32)

    deg = _deg_call(dst_deg, ones128, zrow)
    degf = (deg[0] + deg[1]).reshape(NP) + 1.0
    dinv = jnp.where(jnp.arange(NP) < N, lax.rsqrt(degf),
                     0.0).reshape(NP, 1)

    def lin(p):
        return p.reshape(NP, PW)

    def packsum(a):
        return (a[0] + a[1]).reshape(NP // 2, 128)

    g1 = _tc1(dinv, x_p, W1)
    acc1 = _prop(lin(g1), srcP, dstP, zwide)
    g2 = _tc_mid(packsum(acc1), g1, dinv, b1.reshape(1, 64), W2, 64)
    acc2 = _prop(lin(g2), srcP, dstP, zwide)
    g3 = _tc_mid(packsum(acc2), g2, dinv, b2.reshape(1, 64), Wmu, 16)
    acc3 = _prop(lin(g3), srcP, dstP, zwide)
    mu_lo, mu_hi = _tc_mu(packsum(acc3), g3, dinv, bmu.reshape(1, 16))
    mu = jnp.concatenate([mu_lo, mu_hi], axis=0)
    return _tc_dec(mu)


# speedup vs baseline: 1.4046x; 1.4046x over previous
import jax
import jax.numpy as jnp
from jax import lax
from jax.experimental import pallas as pl
from jax.experimental.pallas import tpu as pltpu
from jax.experimental.pallas import tpu_sc as plsc

N = 10000
NP = 10240
NC, NS = 2, 16
NWORK = NC * NS
CHUNK = 128
CH = 80
EPT = CH * CHUNK
ETOT = NWORK * EPT
RPT = NP // NS
PW = 64

NBUF = 8
LAG = 4

BF = jnp.bfloat16


def _sc_mesh():
    return plsc.VectorSubcoreMesh(core_axis_name="c", subcore_axis_name="s",
                                  num_cores=NC, num_subcores=NS)


def _deg_body(dst_hbm, ones_hbm, zeros_hbm, out_hbm, dst_v, ones_v, acc,
              *sems):
    c = lax.axis_index("c")
    s = lax.axis_index("s")
    lin = c * NS + s
    pltpu.sync_copy(dst_hbm.at[lin], dst_v)
    pltpu.sync_copy(ones_hbm, ones_v)
    pltpu.sync_copy(zeros_hbm, acc.at[pl.ds(s * RPT, RPT)])
    plsc.subcore_barrier()
    dsc = {}
    for ch in range(CH):
        if ch >= 4:
            dsc[ch - 4].wait()
        dsc[ch] = pltpu.async_copy(ones_v, acc.at[dst_v.at[ch]],
                                   sems[ch % 4], add=True)
    for ch in range(CH - 4, CH):
        dsc[ch].wait()
    plsc.subcore_barrier()
    for j in range(RPT // CHUNK):
        pltpu.sync_copy(acc.at[pl.ds(s * RPT + j * CHUNK, CHUNK)],
                        out_hbm.at[c, s * (RPT // CHUNK) + j])


_sc_cache = {}


def _deg_call(*args):
    if "deg" not in _sc_cache:
        _sc_cache["deg"] = pl.kernel(
            _deg_body,
            out_type=jax.ShapeDtypeStruct((NC, NP // CHUNK, CHUNK),
                                          jnp.float32),
            mesh=_sc_mesh(),
            scratch_types=[
                pltpu.VMEM((CH, CHUNK), jnp.int32),
                pltpu.VMEM((CHUNK,), jnp.float32),
                pltpu.VMEM_SHARED((NP,), jnp.float32),
            ] + [pltpu.SemaphoreType.DMA] * 4,
        )
    return _sc_cache["deg"](*args)


def _prop_body(g_hbm, src_hbm, dst_hbm, zeros_hbm, out_hbm,
               src_v, dst_v, rows, acc, *sems):
    gsem = sems[:NBUF]
    ssem = sems[NBUF:]
    c = lax.axis_index("c")
    s = lax.axis_index("s")
    lin = c * NS + s
    pltpu.sync_copy(src_hbm.at[lin], src_v)
    pltpu.sync_copy(dst_hbm.at[lin], dst_v)
    pltpu.sync_copy(zeros_hbm, acc.at[pl.ds(s * RPT, RPT)])
    plsc.subcore_barrier()
    dg = {}
    dsc = {}
    for j in range(LAG):
        dg[j] = pltpu.async_copy(g_hbm.at[src_v.at[j]],
                                 rows.at[j % NBUF], gsem[j % NBUF])
    for ch in range(CH):
        b = ch % NBUF
        dg[ch].wait()
        dsc[ch] = pltpu.async_copy(rows.at[b], acc.at[dst_v.at[ch]],
                                   ssem[b], add=True)
        g = ch + LAG
        if g < CH:
            bg = g % NBUF
            if g - NBUF >= 0:
                dsc[g - NBUF].wait()
            dg[g] = pltpu.async_copy(g_hbm.at[src_v.at[g]],
                                     rows.at[bg], gsem[bg])
    for ch in range(CH - NBUF, CH):
        dsc[ch].wait()
    plsc.subcore_barrier()
    pltpu.sync_copy(acc.at[pl.ds(s * RPT, RPT)],
                    out_hbm.at[c, pl.ds(s * RPT, RPT)])


def _prop(*args):
    if "prop" not in _sc_cache:
        _sc_cache["prop"] = pl.kernel(
            _prop_body,
            out_type=jax.ShapeDtypeStruct((NC, NP, PW), BF),
            mesh=_sc_mesh(),
            compiler_params=pltpu.CompilerParams(use_tc_tiling_on_sc=False),
            scratch_types=[
                pltpu.VMEM((CH, CHUNK), jnp.int32),
                pltpu.VMEM((CH, CHUNK), jnp.int32),
                pltpu.VMEM((NBUF, CHUNK, PW), BF),
                pltpu.VMEM_SHARED((NP, PW), BF),
            ] + [pltpu.SemaphoreType.DMA] * (2 * NBUF),
        )
    return _sc_cache["prop"](*args)


_BLK = 1024
_GRID = NP // _BLK


def _tc1_body(dinv_ref, x_ref, w_ref, g_ref):
    g_ref[...] = (dinv_ref[...] * jnp.dot(
        x_ref[...], w_ref[...],
        preferred_element_type=jnp.float32)).astype(BF)


def _tc_mid_body(dn, acc0, acc1, g_ref, dinv_ref, b_ref, w_ref, out_ref):
    d = w_ref.shape[0]
    dinv = dinv_ref[...]
    gsum = (acc0[:, :d].astype(jnp.float32) + acc1[:, :d].astype(jnp.float32)
            + g_ref[:, :d].astype(jnp.float32))
    h = jnp.maximum(dinv * gsum + b_ref[...], 0.0)
    out_ref[:, :dn] = (dinv * jnp.dot(
        h, w_ref[...], preferred_element_type=jnp.float32)).astype(BF)
    if dn < PW:
        out_ref[:, dn:] = jnp.zeros((_BLK, PW - dn), BF)


def _tc_mu_body(acc0, acc1, g_ref, dinv_ref, b_ref, mu_ref):
    gsum = (acc0[:, :16].astype(jnp.float32)
            + acc1[:, :16].astype(jnp.float32)
            + g_ref[:, :16].astype(jnp.float32))
    mu_ref[...] = dinv_ref[...] * gsum + b_ref[...]


_DBLK = 400
_DGRID = N // _DBLK


def _tc_dec_body(mu_i, mu_j, out_ref):
    logits = lax.dot_general(mu_i[...], mu_j[...],
                             (((1,), (1,)), ((), ())),
                             preferred_element_type=jnp.float32)
    out_ref[...] = jax.nn.sigmoid(logits)


def _row_spec(d):
    return pl.BlockSpec((_BLK, d), lambda i: (i, 0))


def _full_spec(shape):
    return pl.BlockSpec(shape, lambda i: tuple(0 for _ in shape))


def _tc1(dinv, x_p, W1):
    return pl.pallas_call(
        _tc1_body,
        grid=(_GRID,),
        in_specs=[_row_spec(1), _row_spec(128), _full_spec((128, 64))],
        out_specs=_row_spec(PW),
        out_shape=jax.ShapeDtypeStruct((NP, PW), BF),
    )(dinv, x_p, W1)


def _tc_mid(acc, g, dinv, b, W, dn):
    d = W.shape[0]
    return pl.pallas_call(
        lambda *refs: _tc_mid_body(dn, *refs),
        grid=(_GRID,),
        in_specs=[_row_spec(PW), _row_spec(PW), _row_spec(PW), _row_spec(1),
                  _full_spec((1, d)), _full_spec((d, dn))],
        out_specs=_row_spec(PW),
        out_shape=jax.ShapeDtypeStruct((NP, PW), BF),
    )(acc[0], acc[1], g, dinv, b, W)


def _tc_mu(acc, g, dinv, b):
    return pl.pallas_call(
        _tc_mu_body,
        grid=(_GRID,),
        in_specs=[_row_spec(PW), _row_spec(PW), _row_spec(PW), _row_spec(1),
                  _full_spec((1, 16))],
        out_specs=_row_spec(16),
        out_shape=jax.ShapeDtypeStruct((NP, 16), jnp.float32),
    )(acc[0], acc[1], g, dinv, b)


def _tc_dec(mu):
    return pl.pallas_call(
        _tc_dec_body,
        grid=(_DGRID,),
        in_specs=[pl.BlockSpec((_DBLK, 16), lambda i: (i, 0)),
                  pl.BlockSpec((N, 16), lambda i: (0, 0))],
        out_specs=pl.BlockSpec((_DBLK, N), lambda i: (i, 0)),
        out_shape=jax.ShapeDtypeStruct((N, N), jnp.float32),
    )(mu, mu)


def kernel(x, edge_index, W1, b1, W2, b2, Wmu, bmu, Wlv, blv):
    del Wlv, blv
    src = edge_index[0]
    dst = edge_index[1]
    pad = ETOT - src.shape[0]
    fill = jnp.full((pad,), N, dtype=jnp.int32)
    src_p = jnp.concatenate([src, fill]).reshape(NWORK, CH, CHUNK)
    dst_p = jnp.concatenate([dst, fill]).reshape(NWORK, CH, CHUNK)
    x_p = jnp.concatenate(
        [x, jnp.zeros((NP - N, x.shape[1]), jnp.float32)], axis=0)

    ones128 = jnp.ones((CHUNK,), jnp.float32)
    zrow = jnp.zeros((RPT,), jnp.float32)
    zwide = jnp.zeros((RPT, PW), BF)

    deg = _deg_call(dst_p, ones128, zrow)
    degf = (deg[0] + deg[1]).reshape(NP) + 1.0
    dinv = jnp.where(jnp.arange(NP) < N, lax.rsqrt(degf),
                     0.0).reshape(NP, 1)

    g1 = _tc1(dinv, x_p, W1)
    acc1 = _prop(g1, src_p, dst_p, zwide)
    g2 = _tc_mid(acc1, g1, dinv, b1.reshape(1, 64), W2, 64)
    acc2 = _prop(g2, src_p, dst_p, zwide)
    g3 = _tc_mid(acc2, g2, dinv, b2.reshape(1, 64), Wmu, 16)
    acc3 = _prop(g3, src_p, dst_p, zwide)
    mu = _tc_mu(acc3, g3, dinv, bmu.reshape(1, 16))
    return _tc_dec(mu)


# speedup vs baseline: 1.4068x; 1.0016x over previous
import jax
import jax.numpy as jnp
from jax import lax
from jax.experimental import pallas as pl
from jax.experimental.pallas import tpu as pltpu
from jax.experimental.pallas import tpu_sc as plsc

N = 10000
NP = 10240
NC, NS = 2, 16
NWORK = NC * NS
CHUNK = 128
CH = 80
EPT = CH * CHUNK
ETOT = NWORK * EPT
RPT = NP // NS
PW = 64

NBUF = 8
LAG = 4

BF = jnp.bfloat16


def _sc_mesh():
    return plsc.VectorSubcoreMesh(core_axis_name="c", subcore_axis_name="s",
                                  num_cores=NC, num_subcores=NS)


def _deg_body(dst_hbm, ones_hbm, zeros_hbm, out_hbm, dst_v, ones_v, acc,
              *sems):
    c = lax.axis_index("c")
    s = lax.axis_index("s")
    lin = c * NS + s
    pltpu.sync_copy(dst_hbm.at[lin], dst_v)
    pltpu.sync_copy(ones_hbm, ones_v)
    pltpu.sync_copy(zeros_hbm, acc.at[pl.ds(s * RPT, RPT)])
    plsc.subcore_barrier()
    dsc = {}
    for ch in range(CH):
        if ch >= 4:
            dsc[ch - 4].wait()
        dsc[ch] = pltpu.async_copy(ones_v, acc.at[dst_v.at[ch]],
                                   sems[ch % 4], add=True)
    for ch in range(CH - 4, CH):
        dsc[ch].wait()
    plsc.subcore_barrier()
    for j in range(RPT // CHUNK):
        pltpu.sync_copy(acc.at[pl.ds(s * RPT + j * CHUNK, CHUNK)],
                        out_hbm.at[c, s * (RPT // CHUNK) + j])


_sc_cache = {}


def _deg_call(*args):
    if "deg" not in _sc_cache:
        _sc_cache["deg"] = pl.kernel(
            _deg_body,
            out_type=jax.ShapeDtypeStruct((NC, NP // CHUNK, CHUNK),
                                          jnp.float32),
            mesh=_sc_mesh(),
            scratch_types=[
                pltpu.VMEM((CH, CHUNK), jnp.int32),
                pltpu.VMEM((CHUNK,), jnp.float32),
                pltpu.VMEM_SHARED((NP,), jnp.float32),
            ] + [pltpu.SemaphoreType.DMA] * 4,
        )
    return _sc_cache["deg"](*args)


def _prop_body(g_hbm, src_hbm, dst_hbm, zeros_hbm, out_hbm,
               src_v, dst_v, rows, acc, *sems):
    gsem = sems[:NBUF]
    ssem = sems[NBUF:]
    c = lax.axis_index("c")
    s = lax.axis_index("s")
    lin = c * NS + s
    pltpu.sync_copy(src_hbm.at[lin], src_v)
    pltpu.sync_copy(dst_hbm.at[lin], dst_v)
    pltpu.sync_copy(zeros_hbm, acc.at[pl.ds(s * RPT, RPT)])
    plsc.subcore_barrier()
    dg = {}
    dsc = {}
    for j in range(LAG):
        dg[j] = pltpu.async_copy(g_hbm.at[src_v.at[j]],
                                 rows.at[j % NBUF], gsem[j % NBUF])
    for ch in range(CH):
        b = ch % NBUF
        dg[ch].wait()
        dsc[ch] = pltpu.async_copy(rows.at[b], acc.at[dst_v.at[ch]],
                                   ssem[b], add=True)
        g = ch + LAG
        if g < CH:
            bg = g % NBUF
            if g - NBUF >= 0:
                dsc[g - NBUF].wait()
            dg[g] = pltpu.async_copy(g_hbm.at[src_v.at[g]],
                                     rows.at[bg], gsem[bg])
    for ch in range(CH - NBUF, CH):
        dsc[ch].wait()
    plsc.subcore_barrier()
    pltpu.sync_copy(acc.at[pl.ds(s * RPT, RPT)],
                    out_hbm.at[c, pl.ds(s * RPT, RPT)])


def _prop(*args):
    if "prop" not in _sc_cache:
        _sc_cache["prop"] = pl.kernel(
            _prop_body,
            out_type=jax.ShapeDtypeStruct((NC, NP, PW), BF),
            mesh=_sc_mesh(),
            compiler_params=pltpu.CompilerParams(use_tc_tiling_on_sc=False),
            scratch_types=[
                pltpu.VMEM((CH, CHUNK), jnp.int32),
                pltpu.VMEM((CH, CHUNK), jnp.int32),
                pltpu.VMEM((NBUF, CHUNK, PW), BF),
                pltpu.VMEM_SHARED((NP, PW), BF),
            ] + [pltpu.SemaphoreType.DMA] * (2 * NBUF),
        )
    return _sc_cache["prop"](*args)


_BLK = 1024
_GRID = NP // _BLK


def _tc1_body(dinv_ref, x_ref, w_ref, g_ref):
    g_ref[...] = (dinv_ref[...] * jnp.dot(
        x_ref[...], w_ref[...],
        preferred_element_type=jnp.float32)).astype(BF)


def _tc_mid_body(dn, acc0, acc1, g_ref, dinv_ref, b_ref, w_ref, out_ref):
    d = w_ref.shape[0]
    dinv = dinv_ref[...]
    gsum = (acc0[:, :d].astype(jnp.float32) + acc1[:, :d].astype(jnp.float32)
            + g_ref[:, :d].astype(jnp.float32))
    h = jnp.maximum(dinv * gsum + b_ref[...], 0.0)
    out_ref[:, :dn] = (dinv * jnp.dot(
        h, w_ref[...], preferred_element_type=jnp.float32)).astype(BF)
    if dn < PW:
        out_ref[:, dn:] = jnp.zeros((_BLK, PW - dn), BF)


def _tc_mu_body(acc0, acc1, g_ref, dinv_ref, b_ref, mu_ref):
    gsum = (acc0[:, :16].astype(jnp.float32)
            + acc1[:, :16].astype(jnp.float32)
            + g_ref[:, :16].astype(jnp.float32))
    mu_ref[...] = (dinv_ref[...] * gsum + b_ref[...]).astype(BF)


_DBLK = 400
_DGRID = N // _DBLK


def _tc_dec_body(mu_i, mu_j, out_ref):
    logits = lax.dot_general(mu_i[...], mu_j[...],
                             (((1,), (1,)), ((), ())),
                             preferred_element_type=jnp.float32)
    out_ref[...] = jax.nn.sigmoid(logits)


def _row_spec(d):
    return pl.BlockSpec((_BLK, d), lambda i: (i, 0))


def _full_spec(shape):
    return pl.BlockSpec(shape, lambda i: tuple(0 for _ in shape))


def _tc1(dinv, x_p, W1):
    return pl.pallas_call(
        _tc1_body,
        grid=(_GRID,),
        in_specs=[_row_spec(1), _row_spec(128), _full_spec((128, 64))],
        out_specs=_row_spec(PW),
        out_shape=jax.ShapeDtypeStruct((NP, PW), BF),
    )(dinv, x_p, W1)


def _tc_mid(acc, g, dinv, b, W, dn):
    d = W.shape[0]
    return pl.pallas_call(
        lambda *refs: _tc_mid_body(dn, *refs),
        grid=(_GRID,),
        in_specs=[_row_spec(PW), _row_spec(PW), _row_spec(PW), _row_spec(1),
                  _full_spec((1, d)), _full_spec((d, dn))],
        out_specs=_row_spec(PW),
        out_shape=jax.ShapeDtypeStruct((NP, PW), BF),
    )(acc[0], acc[1], g, dinv, b, W)


def _tc_mu(acc, g, dinv, b):
    return pl.pallas_call(
        _tc_mu_body,
        grid=(_GRID,),
        in_specs=[_row_spec(PW), _row_spec(PW), _row_spec(PW), _row_spec(1),
                  _full_spec((1, 16))],
        out_specs=_row_spec(16),
        out_shape=jax.ShapeDtypeStruct((NP, 16), BF),
    )(acc[0], acc[1], g, dinv, b)


def _tc_dec(mu):
    return pl.pallas_call(
        _tc_dec_body,
        grid=(_DGRID,),
        in_specs=[pl.BlockSpec((_DBLK, 16), lambda i: (i, 0)),
                  pl.BlockSpec((N, 16), lambda i: (0, 0))],
        out_specs=pl.BlockSpec((_DBLK, N), lambda i: (i, 0)),
        out_shape=jax.ShapeDtypeStruct((N, N), jnp.float32),
    )(mu, mu)


def kernel(x, edge_index, W1, b1, W2, b2, Wmu, bmu, Wlv, blv):
    del Wlv, blv
    src = edge_index[0]
    dst = edge_index[1]
    pad = ETOT - src.shape[0]
    fill = jnp.full((pad,), N, dtype=jnp.int32)
    src_p = jnp.concatenate([src, fill]).reshape(NWORK, CH, CHUNK)
    dst_p = jnp.concatenate([dst, fill]).reshape(NWORK, CH, CHUNK)
    x_p = jnp.concatenate(
        [x, jnp.zeros((NP - N, x.shape[1]), jnp.float32)], axis=0)

    ones128 = jnp.ones((CHUNK,), jnp.float32)
    zrow = jnp.zeros((RPT,), jnp.float32)
    zwide = jnp.zeros((RPT, PW), BF)

    deg = _deg_call(dst_p, ones128, zrow)
    degf = (deg[0] + deg[1]).reshape(NP) + 1.0
    dinv = jnp.where(jnp.arange(NP) < N, lax.rsqrt(degf),
                     0.0).reshape(NP, 1)

    g1 = _tc1(dinv, x_p, W1)
    acc1 = _prop(g1, src_p, dst_p, zwide)
    g2 = _tc_mid(acc1, g1, dinv, b1.reshape(1, 64), W2, 64)
    acc2 = _prop(g2, src_p, dst_p, zwide)
    g3 = _tc_mid(acc2, g2, dinv, b2.reshape(1, 64), Wmu, 16)
    acc3 = _prop(g3, src_p, dst_p, zwide)
    mu = _tc_mu(acc3, g3, dinv, bmu.reshape(1, 16))
    return _tc_dec(mu)


# speedup vs baseline: 1.5458x; 1.0989x over previous
import jax
import jax.numpy as jnp
from jax import lax
from jax.experimental import pallas as pl
from jax.experimental.pallas import tpu as pltpu
from jax.experimental.pallas import tpu_sc as plsc

N = 10000
NP = 10240
NC, NS = 2, 16
NWORK = NC * NS
CHUNK = 128
CH = 80
EPT = CH * CHUNK
ETOT = NWORK * EPT
RPT = NP // NS
PW = 64

NBUF = 8
LAG = 4

BF = jnp.bfloat16


def _sc_mesh():
    return plsc.VectorSubcoreMesh(core_axis_name="c", subcore_axis_name="s",
                                  num_cores=NC, num_subcores=NS)


def _deg_body(dst_hbm, ones_hbm, zeros_hbm, out_hbm, dst_v, ones_v, acc,
              *sems):
    c = lax.axis_index("c")
    s = lax.axis_index("s")
    lin = c * NS + s
    pltpu.sync_copy(dst_hbm.at[lin], dst_v)
    pltpu.sync_copy(ones_hbm, ones_v)
    pltpu.sync_copy(zeros_hbm, acc.at[pl.ds(s * RPT, RPT)])
    plsc.subcore_barrier()
    dsc = {}
    for ch in range(CH):
        if ch >= 4:
            dsc[ch - 4].wait()
        dsc[ch] = pltpu.async_copy(ones_v, acc.at[dst_v.at[ch]],
                                   sems[ch % 4], add=True)
    for ch in range(CH - 4, CH):
        dsc[ch].wait()
    plsc.subcore_barrier()
    for j in range(RPT // CHUNK):
        pltpu.sync_copy(acc.at[pl.ds(s * RPT + j * CHUNK, CHUNK)],
                        out_hbm.at[c, s * (RPT // CHUNK) + j])


_sc_cache = {}


def _deg_call(*args):
    if "deg" not in _sc_cache:
        _sc_cache["deg"] = pl.kernel(
            _deg_body,
            out_type=jax.ShapeDtypeStruct((NC, NP // CHUNK, CHUNK),
                                          jnp.float32),
            mesh=_sc_mesh(),
            scratch_types=[
                pltpu.VMEM((CH, CHUNK), jnp.int32),
                pltpu.VMEM((CHUNK,), jnp.float32),
                pltpu.VMEM_SHARED((NP,), jnp.float32),
            ] + [pltpu.SemaphoreType.DMA] * 4,
        )
    return _sc_cache["deg"](*args)


def _prop_body(g_hbm, src_hbm, dst_hbm, zeros_hbm, out_hbm,
               src_v, dst_v, rows, acc, *sems):
    gsem = sems[:NBUF]
    ssem = sems[NBUF:]
    c = lax.axis_index("c")
    s = lax.axis_index("s")
    lin = c * NS + s
    pltpu.sync_copy(src_hbm.at[lin], src_v)
    pltpu.sync_copy(dst_hbm.at[lin], dst_v)
    pltpu.sync_copy(zeros_hbm, acc.at[pl.ds(s * RPT, RPT)])
    plsc.subcore_barrier()
    dg = {}
    dsc = {}
    for j in range(LAG):
        dg[j] = pltpu.async_copy(g_hbm.at[src_v.at[j]],
                                 rows.at[j % NBUF], gsem[j % NBUF])
    for ch in range(CH):
        b = ch % NBUF
        dg[ch].wait()
        dsc[ch] = pltpu.async_copy(rows.at[b], acc.at[dst_v.at[ch]],
                                   ssem[b], add=True)
        g = ch + LAG
        if g < CH:
            bg = g % NBUF
            if g - NBUF >= 0:
                dsc[g - NBUF].wait()
            dg[g] = pltpu.async_copy(g_hbm.at[src_v.at[g]],
                                     rows.at[bg], gsem[bg])
    for ch in range(CH - NBUF, CH):
        dsc[ch].wait()
    plsc.subcore_barrier()
    pltpu.sync_copy(acc.at[pl.ds(s * RPT, RPT)],
                    out_hbm.at[c, pl.ds(s * RPT, RPT)])


def _prop(pw, *args):
    key = ("prop", pw)
    if key not in _sc_cache:
        _sc_cache[key] = pl.kernel(
            _prop_body,
            out_type=jax.ShapeDtypeStruct((NC, NP, pw), BF),
            mesh=_sc_mesh(),
            compiler_params=pltpu.CompilerParams(use_tc_tiling_on_sc=False),
            scratch_types=[
                pltpu.VMEM((CH, CHUNK), jnp.int32),
                pltpu.VMEM((CH, CHUNK), jnp.int32),
                pltpu.VMEM((NBUF, CHUNK, pw), BF),
                pltpu.VMEM_SHARED((NP, pw), BF),
            ] + [pltpu.SemaphoreType.DMA] * (2 * NBUF),
        )
    return _sc_cache[key](*args)


_BLK = 1024
_GRID = NP // _BLK


def _tc1_body(dinv_ref, x_ref, w_ref, g_ref):
    g_ref[...] = (dinv_ref[...] * jnp.dot(
        x_ref[...], w_ref[...],
        preferred_element_type=jnp.float32)).astype(BF)


def _tc_mid_body(dn, ow, acc0, acc1, g_ref, dinv_ref, b_ref, w_ref, out_ref):
    d = w_ref.shape[0]
    dinv = dinv_ref[...]
    gsum = (acc0[:, :d].astype(jnp.float32) + acc1[:, :d].astype(jnp.float32)
            + g_ref[:, :d].astype(jnp.float32))
    h = jnp.maximum(dinv * gsum + b_ref[...], 0.0)
    out_ref[:, :dn] = (dinv * jnp.dot(
        h, w_ref[...], preferred_element_type=jnp.float32)).astype(BF)
    if dn < ow:
        out_ref[:, dn:] = jnp.zeros((_BLK, ow - dn), BF)


def _tc_mu_body(acc0, acc1, g_ref, dinv_ref, b_ref, mu_ref):
    gsum = (acc0[:, :16].astype(jnp.float32)
            + acc1[:, :16].astype(jnp.float32)
            + g_ref[:, :16].astype(jnp.float32))
    mu_ref[...] = (dinv_ref[...] * gsum + b_ref[...]).astype(BF)


_DBLK = 400
_DGRID = N // _DBLK


def _tc_dec_body(mu_i, mu_j, out_ref):
    logits = lax.dot_general(mu_i[...], mu_j[...],
                             (((1,), (1,)), ((), ())),
                             preferred_element_type=jnp.float32)
    out_ref[...] = jax.nn.sigmoid(logits)


def _row_spec(d):
    return pl.BlockSpec((_BLK, d), lambda i: (i, 0))


def _full_spec(shape):
    return pl.BlockSpec(shape, lambda i: tuple(0 for _ in shape))


def _tc1(dinv, x_p, W1):
    return pl.pallas_call(
        _tc1_body,
        grid=(_GRID,),
        in_specs=[_row_spec(1), _row_spec(128), _full_spec((128, 64))],
        out_specs=_row_spec(PW),
        out_shape=jax.ShapeDtypeStruct((NP, PW), BF),
    )(dinv, x_p, W1)


def _tc_mid(acc, g, dinv, b, W, dn, ow):
    d = W.shape[0]
    gw = g.shape[1]
    return pl.pallas_call(
        lambda *refs: _tc_mid_body(dn, ow, *refs),
        grid=(_GRID,),
        in_specs=[_row_spec(gw), _row_spec(gw), _row_spec(gw), _row_spec(1),
                  _full_spec((1, d)), _full_spec((d, dn))],
        out_specs=_row_spec(ow),
        out_shape=jax.ShapeDtypeStruct((NP, ow), BF),
    )(acc[0], acc[1], g, dinv, b, W)


def _tc_mu(acc, g, dinv, b):
    gw = g.shape[1]
    return pl.pallas_call(
        _tc_mu_body,
        grid=(_GRID,),
        in_specs=[_row_spec(gw), _row_spec(gw), _row_spec(gw), _row_spec(1),
                  _full_spec((1, 16))],
        out_specs=_row_spec(16),
        out_shape=jax.ShapeDtypeStruct((NP, 16), BF),
    )(acc[0], acc[1], g, dinv, b)


def _tc_dec(mu):
    return pl.pallas_call(
        _tc_dec_body,
        grid=(_DGRID,),
        in_specs=[pl.BlockSpec((_DBLK, 16), lambda i: (i, 0)),
                  pl.BlockSpec((N, 16), lambda i: (0, 0))],
        out_specs=pl.BlockSpec((_DBLK, N), lambda i: (i, 0)),
        out_shape=jax.ShapeDtypeStruct((N, N), jnp.float32),
    )(mu, mu)


def kernel(x, edge_index, W1, b1, W2, b2, Wmu, bmu, Wlv, blv):
    del Wlv, blv
    src = edge_index[0]
    dst = edge_index[1]
    pad = ETOT - src.shape[0]
    fill = jnp.full((pad,), N, dtype=jnp.int32)
    src_p = jnp.concatenate([src, fill]).reshape(NWORK, CH, CHUNK)
    dst_p = jnp.concatenate([dst, fill]).reshape(NWORK, CH, CHUNK)
    x_p = jnp.concatenate(
        [x, jnp.zeros((NP - N, x.shape[1]), jnp.float32)], axis=0)

    ones128 = jnp.ones((CHUNK,), jnp.float32)
    zrow = jnp.zeros((RPT,), jnp.float32)
    z64 = jnp.zeros((RPT, 64), BF)
    z32 = jnp.zeros((RPT, 32), BF)

    deg = _deg_call(dst_p, ones128, zrow)
    degf = (deg[0] + deg[1]).reshape(NP) + 1.0
    dinv = jnp.where(jnp.arange(NP) < N, lax.rsqrt(degf),
                     0.0).reshape(NP, 1)

    g1 = _tc1(dinv, x_p, W1)
    acc1 = _prop(64, g1, src_p, dst_p, z64)
    g2 = _tc_mid(acc1, g1, dinv, b1.reshape(1, 64), W2, 64, 64)
    acc2 = _prop(64, g2, src_p, dst_p, z64)
    g3 = _tc_mid(acc2, g2, dinv, b2.reshape(1, 64), Wmu, 16, 32)
    acc3 = _prop(32, g3, src_p, dst_p, z32)
    mu = _tc_mu(acc3, g3, dinv, bmu.reshape(1, 16))
    return _tc_dec(mu)


# speedup vs baseline: 1.5512x; 1.0034x over previous
import jax
import jax.numpy as jnp
from jax import lax
from jax.experimental import pallas as pl
from jax.experimental.pallas import tpu as pltpu
from jax.experimental.pallas import tpu_sc as plsc

N = 10000
NP = 10240
NC, NS = 2, 16
NWORK = NC * NS
CHUNK = 128
CH = 80
EPT = CH * CHUNK
ETOT = NWORK * EPT
RPT = NP // NS
PW = 64

NBUF = 8
LAG = 4

BF = jnp.bfloat16


def _sc_mesh():
    return plsc.VectorSubcoreMesh(core_axis_name="c", subcore_axis_name="s",
                                  num_cores=NC, num_subcores=NS)


def _deg_body(dst_hbm, ones_hbm, zeros_hbm, out_hbm, dst_v, ones_v, acc,
              *sems):
    c = lax.axis_index("c")
    s = lax.axis_index("s")
    lin = c * NS + s
    pltpu.sync_copy(dst_hbm.at[lin], dst_v)
    pltpu.sync_copy(ones_hbm, ones_v)
    pltpu.sync_copy(zeros_hbm, acc.at[pl.ds(s * RPT, RPT)])
    plsc.subcore_barrier()
    dsc = {}
    for ch in range(CH):
        if ch >= 4:
            dsc[ch - 4].wait()
        dsc[ch] = pltpu.async_copy(ones_v, acc.at[dst_v.at[ch]],
                                   sems[ch % 4], add=True)
    for ch in range(CH - 4, CH):
        dsc[ch].wait()
    plsc.subcore_barrier()
    for j in range(RPT // CHUNK):
        pltpu.sync_copy(acc.at[pl.ds(s * RPT + j * CHUNK, CHUNK)],
                        out_hbm.at[c, s * (RPT // CHUNK) + j])


_sc_cache = {}


def _deg_call(*args):
    if "deg" not in _sc_cache:
        _sc_cache["deg"] = pl.kernel(
            _deg_body,
            out_type=jax.ShapeDtypeStruct((NC, NP // CHUNK, CHUNK),
                                          jnp.float32),
            mesh=_sc_mesh(),
            compiler_params=pltpu.CompilerParams(use_tc_tiling_on_sc=False),
            scratch_types=[
                pltpu.VMEM((CH, CHUNK), jnp.int32),
                pltpu.VMEM((CHUNK,), jnp.float32),
                pltpu.VMEM_SHARED((NP,), jnp.float32),
            ] + [pltpu.SemaphoreType.DMA] * 4,
        )
    return _sc_cache["deg"](*args)


def _prop_body(g_hbm, src_hbm, dst_hbm, zeros_hbm, out_hbm,
               src_v, dst_v, rows, acc, *sems):
    gsem = sems[:NBUF]
    ssem = sems[NBUF:]
    c = lax.axis_index("c")
    s = lax.axis_index("s")
    lin = c * NS + s
    pltpu.sync_copy(src_hbm.at[lin], src_v)
    pltpu.sync_copy(dst_hbm.at[lin], dst_v)
    pltpu.sync_copy(zeros_hbm, acc.at[pl.ds(s * RPT, RPT)])
    plsc.subcore_barrier()
    dg = {}
    dsc = {}
    for j in range(LAG):
        dg[j] = pltpu.async_copy(g_hbm.at[src_v.at[j]],
                                 rows.at[j % NBUF], gsem[j % NBUF])
    for ch in range(CH):
        b = ch % NBUF
        dg[ch].wait()
        dsc[ch] = pltpu.async_copy(rows.at[b], acc.at[dst_v.at[ch]],
                                   ssem[b], add=True)
        g = ch + LAG
        if g < CH:
            bg = g % NBUF
            if g - NBUF >= 0:
                dsc[g - NBUF].wait()
            dg[g] = pltpu.async_copy(g_hbm.at[src_v.at[g]],
                                     rows.at[bg], gsem[bg])
    for ch in range(CH - NBUF, CH):
        dsc[ch].wait()
    plsc.subcore_barrier()
    pltpu.sync_copy(acc.at[pl.ds(s * RPT, RPT)],
                    out_hbm.at[c, pl.ds(s * RPT, RPT)])


def _prop(pw, *args):
    key = ("prop", pw)
    if key not in _sc_cache:
        _sc_cache[key] = pl.kernel(
            _prop_body,
            out_type=jax.ShapeDtypeStruct((NC, NP, pw), BF),
            mesh=_sc_mesh(),
            compiler_params=pltpu.CompilerParams(use_tc_tiling_on_sc=False),
            scratch_types=[
                pltpu.VMEM((CH, CHUNK), jnp.int32),
                pltpu.VMEM((CH, CHUNK), jnp.int32),
                pltpu.VMEM((NBUF, CHUNK, pw), BF),
                pltpu.VMEM_SHARED((NP, pw), BF),
            ] + [pltpu.SemaphoreType.DMA] * (2 * NBUF),
        )
    return _sc_cache[key](*args)


_BLK = 1024
_GRID = NP // _BLK


def _tc1_body(dinv_ref, x_ref, w_ref, g_ref):
    g_ref[...] = (dinv_ref[...] * jnp.dot(
        x_ref[...], w_ref[...],
        preferred_element_type=jnp.float32)).astype(BF)


def _tc_mid_body(dn, ow, acc0, acc1, g_ref, dinv_ref, b_ref, w_ref, out_ref):
    d = w_ref.shape[0]
    dinv = dinv_ref[...]
    gsum = (acc0[:, :d].astype(jnp.float32) + acc1[:, :d].astype(jnp.float32)
            + g_ref[:, :d].astype(jnp.float32))
    h = jnp.maximum(dinv * gsum + b_ref[...], 0.0)
    out_ref[:, :dn] = (dinv * jnp.dot(
        h, w_ref[...], preferred_element_type=jnp.float32)).astype(BF)
    if dn < ow:
        out_ref[:, dn:] = jnp.zeros((_BLK, ow - dn), BF)


def _tc_mu_body(acc0, acc1, g_ref, dinv_ref, b_ref, mu_ref):
    gsum = (acc0[:, :16].astype(jnp.float32)
            + acc1[:, :16].astype(jnp.float32)
            + g_ref[:, :16].astype(jnp.float32))
    mu_ref[...] = (dinv_ref[...] * gsum + b_ref[...]).astype(BF)


_DBLK = 400
_DGRID = N // _DBLK


def _tc_dec_body(mu_i, mu_j, out_ref):
    logits = lax.dot_general(mu_i[...], mu_j[...],
                             (((1,), (1,)), ((), ())),
                             preferred_element_type=jnp.float32)
    out_ref[...] = jax.nn.sigmoid(logits)


def _row_spec(d):
    return pl.BlockSpec((_BLK, d), lambda i: (i, 0))


def _full_spec(shape):
    return pl.BlockSpec(shape, lambda i: tuple(0 for _ in shape))


def _tc1(dinv, x_p, W1):
    return pl.pallas_call(
        _tc1_body,
        grid=(_GRID,),
        in_specs=[_row_spec(1), _row_spec(128), _full_spec((128, 64))],
        out_specs=_row_spec(PW),
        out_shape=jax.ShapeDtypeStruct((NP, PW), BF),
    )(dinv, x_p, W1)


def _tc_mid(acc, g, dinv, b, W, dn, ow):
    d = W.shape[0]
    gw = g.shape[1]
    return pl.pallas_call(
        lambda *refs: _tc_mid_body(dn, ow, *refs),
        grid=(_GRID,),
        in_specs=[_row_spec(gw), _row_spec(gw), _row_spec(gw), _row_spec(1),
                  _full_spec((1, d)), _full_spec((d, dn))],
        out_specs=_row_spec(ow),
        out_shape=jax.ShapeDtypeStruct((NP, ow), BF),
    )(acc[0], acc[1], g, dinv, b, W)


def _tc_mu(acc, g, dinv, b):
    gw = g.shape[1]
    return pl.pallas_call(
        _tc_mu_body,
        grid=(_GRID,),
        in_specs=[_row_spec(gw), _row_spec(gw), _row_spec(gw), _row_spec(1),
                  _full_spec((1, 16))],
        out_specs=_row_spec(16),
        out_shape=jax.ShapeDtypeStruct((NP, 16), BF),
    )(acc[0], acc[1], g, dinv, b)


def _tc_dec(mu):
    return pl.pallas_call(
        _tc_dec_body,
        grid=(_DGRID,),
        in_specs=[pl.BlockSpec((_DBLK, 16), lambda i: (i, 0)),
                  pl.BlockSpec((N, 16), lambda i: (0, 0))],
        out_specs=pl.BlockSpec((_DBLK, N), lambda i: (i, 0)),
        out_shape=jax.ShapeDtypeStruct((N, N), jnp.float32),
    )(mu, mu)


def kernel(x, edge_index, W1, b1, W2, b2, Wmu, bmu, Wlv, blv):
    del Wlv, blv
    src = edge_index[0]
    dst = edge_index[1]
    pad = ETOT - src.shape[0]
    fill = jnp.full((pad,), N, dtype=jnp.int32)
    src_p = jnp.concatenate([src, fill]).reshape(NWORK, CH, CHUNK)
    dst_p = jnp.concatenate([dst, fill]).reshape(NWORK, CH, CHUNK)
    x_p = jnp.concatenate(
        [x, jnp.zeros((NP - N, x.shape[1]), jnp.float32)], axis=0)

    ones128 = jnp.ones((CHUNK,), jnp.float32)
    zrow = jnp.zeros((RPT,), jnp.float32)
    z64 = jnp.zeros((RPT, 64), BF)
    z32 = jnp.zeros((RPT, 32), BF)

    deg = _deg_call(dst_p, ones128, zrow)
    degf = (deg[0] + deg[1]).reshape(NP) + 1.0
    dinv = jnp.where(jnp.arange(NP) < N, lax.rsqrt(degf),
                     0.0).reshape(NP, 1)

    g1 = _tc1(dinv, x_p, W1)
    acc1 = _prop(64, g1, src_p, dst_p, z64)
    g2 = _tc_mid(acc1, g1, dinv, b1.reshape(1, 64), W2, 64, 64)
    acc2 = _prop(64, g2, src_p, dst_p, z64)
    g3 = _tc_mid(acc2, g2, dinv, b2.reshape(1, 64), Wmu, 16, 32)
    acc3 = _prop(32, g3, src_p, dst_p, z32)
    mu = _tc_mu(acc3, g3, dinv, bmu.reshape(1, 16))
    return _tc_dec(mu)
